# Initial kernel scaffold; baseline (speedup 1.0000x reference)
import functools

import jax
import jax.numpy as jnp
from jax import lax
from jax.experimental import pallas as pl
from jax.experimental.pallas import tpu as pltpu
from jax.experimental.pallas import tpu_sc as plsc

N_TOTAL = 4096 * 4096
NC = 2
NS = 16
L = 16
NW = NC * NS
PER_TILE = N_TOTAL // NW
CHUNK = 16384
NCHUNK = PER_TILE // CHUNK
VPC = CHUNK // L
MAGIC = 201326592.0


def _body(w_hbm, par_hbm, out_hbm, buf, par, cnt_t, sum_t, ssq_t, acc,
          sem0, sem1):
    wid = lax.axis_index("s") * NC + lax.axis_index("c")
    base = wid * PER_TILE

    pltpu.sync_copy(par_hbm, par)

    zeros = jnp.zeros((L,), jnp.float32)
    for k in range(4):
        cnt_t[pl.ds(k * L, L)] = zeros
        sum_t[pl.ds(k * L, L)] = zeros
        ssq_t[pl.ds(k * L, L)] = zeros

    ia16 = par[pl.ds(0, L)]
    a16 = par[pl.ds(L, L)]
    thr = par[pl.ds(2 * L, L)]
    lane = par[pl.ds(3 * L, L)]
    ones = jnp.full((L,), 1.0, jnp.float32)
    magic = jnp.full((L,), MAGIC, jnp.float32)
    hi = jnp.full((L,), 16.0, jnp.float32)
    lo = jnp.full((L,), -32.0, jnp.float32)

    def start_dma(g, b):
        sem = sem1 if b else sem0
        return pltpu.async_copy(
            w_hbm.at[pl.ds(base + g * CHUNK, CHUNK)], buf.at[b], sem)

    start_dma(0, 0)
    sdiff = zeros
    near = zeros
    for g in range(NCHUNK):
        b = g % 2
        if g + 1 < NCHUNK:
            start_dma(g + 1, (g + 1) % 2)
        pltpu.make_async_copy(
            w_hbm.at[pl.ds(base + g * CHUNK, CHUNK)], buf.at[b],
            sem1 if b else sem0).wait()
        bufb = buf.at[b]

        @plsc.parallel_loop(0, VPC, step=1, unroll=8, carry=(sdiff, near))
        def chunk_body(i, c):
            sd, nr = c
            w = bufb[pl.ds(i * L, L)]
            t = jnp.minimum(w * ia16, hi)
            t = jnp.maximum(t, lo)
            b16 = (t + magic) - magic
            wq = b16 * a16
            d = jnp.abs(w - wq)
            w2 = w * w
            ai = (b16 + lane).astype(jnp.int32)
            plsc.addupdate_scatter(cnt_t, [ai], ones)
            plsc.addupdate_scatter(sum_t, [ai], w)
            plsc.addupdate_scatter(ssq_t, [ai], w2)
            nr = nr + jnp.where(d < thr, 1.0, 0.0).astype(jnp.float32)
            return (sd + d, nr)

        sdiff, near = chunk_body

    acc[pl.ds(0, L)] = sdiff
    acc[pl.ds(L, L)] = near

    pltpu.sync_copy(cnt_t, out_hbm.at[wid, pl.ds(0, 64)])
    pltpu.sync_copy(sum_t, out_hbm.at[wid, pl.ds(64, 64)])
    pltpu.sync_copy(ssq_t, out_hbm.at[wid, pl.ds(128, 64)])
    pltpu.sync_copy(acc, out_hbm.at[wid, pl.ds(192, 32)])


@jax.jit
def _run(w_flat, params):
    mesh = plsc.VectorSubcoreMesh(core_axis_name="c", subcore_axis_name="s")
    return pl.kernel(
        _body,
        out_type=jax.ShapeDtypeStruct((NW, 224), jnp.float32),
        mesh=mesh,
        scratch_types=[
            pltpu.VMEM((2, CHUNK), jnp.float32),
            pltpu.VMEM((4 * L,), jnp.float32),
            pltpu.VMEM((4 * L,), jnp.float32),
            pltpu.VMEM((4 * L,), jnp.float32),
            pltpu.VMEM((4 * L,), jnp.float32),
            pltpu.VMEM((2 * L,), jnp.float32),
            pltpu.SemaphoreType.DMA,
            pltpu.SemaphoreType.DMA,
        ],
    )(w_flat, params)


def kernel(weights, alpha):
    w = weights.reshape(-1)
    a = alpha.reshape(())
    a_s = lax.stop_gradient(a)
    params = jnp.concatenate([
        jnp.full((L,), 16.0, jnp.float32) / a_s,
        jnp.full((L,), 1.0 / 16.0, jnp.float32) * a_s,
        jnp.full((L,), 0.01, jnp.float32) * a_s,
        jnp.arange(L, dtype=jnp.float32) + 32.0,
    ])
    part = _run(w, params)

    cnt = part[:, 0:64].reshape(NW, 4, L).sum(axis=(0, 2))
    sums = part[:, 64:128].reshape(NW, 4, L).sum(axis=(0, 2))
    ssq = part[:, 128:192].reshape(NW, 4, L).sum(axis=(0, 2))
    sdiff = part[:, 192:208].sum()
    near = part[:, 208:224].sum()

    levels = jnp.arange(-2, 2, dtype=jnp.float32) * a_s
    safe_counts = jnp.maximum(cnt, 1.0)
    means = sums / safe_counts
    mse_per_bin = jnp.where(cnt > 0, (means - levels) ** 2, 0.0)
    var_per_bin = jnp.where(cnt >= 2.0, ssq / safe_counts - means ** 2, 0.0)
    total_mse = jnp.sum(mse_per_bin)
    total_var = jnp.sum(var_per_bin)
    loss = total_mse + total_var

    n = jnp.float32(N_TOTAL)
    sumdiff2 = jnp.sum(ssq - 2.0 * levels * sums + levels * levels * cnt)
    quantization_mse = sumdiff2 / n
    mean_distance = sdiff / n
    max_dist = a_s * 0.5
    effectiveness = jnp.clip(
        100.0 * (1.0 - mean_distance / (max_dist + 1e-12)), 0.0, 100.0)
    near_levels = near / n * 100.0

    return (loss, total_mse, total_var, quantization_mse, mean_distance,
            effectiveness, near_levels)


# speedup vs baseline: 279.2190x; 279.2190x over previous
import functools

import jax
import jax.numpy as jnp
from jax import lax
from jax.experimental import pallas as pl
from jax.experimental.pallas import tpu as pltpu
from jax.experimental.pallas import tpu_sc as plsc

N_TOTAL = 4096 * 4096
NC = 2
NS = 16
L = 16
NW = NC * NS
PER_TILE = N_TOTAL // NW
CHUNK = 16384
NCHUNK = PER_TILE // CHUNK
VPC = CHUNK // L
MAGIC = 201326592.0


def _body(w_hbm, par_hbm, out_hbm, buf0, buf1, par, cnt_t, sum_t, ssq_t, acc,
          sem0, sem1):
    wid = lax.axis_index("s") * NC + lax.axis_index("c")
    base = wid * PER_TILE

    pltpu.sync_copy(par_hbm, par)

    zeros = jnp.zeros((L,), jnp.float32)
    for k in range(4):
        cnt_t[pl.ds(k * L, L)] = zeros
        sum_t[pl.ds(k * L, L)] = zeros
        ssq_t[pl.ds(k * L, L)] = zeros

    ia16 = par[pl.ds(0, L)]
    a16 = par[pl.ds(L, L)]
    thr = par[pl.ds(2 * L, L)]
    lane = par[pl.ds(3 * L, L)]
    ones = jnp.full((L,), 1.0, jnp.float32)
    magic = jnp.full((L,), MAGIC, jnp.float32)
    hi = jnp.full((L,), 16.0, jnp.float32)
    lo = jnp.full((L,), -32.0, jnp.float32)

    def start_dma(g, b):
        return pltpu.async_copy(
            w_hbm.at[pl.ds(base + g * CHUNK, CHUNK)],
            buf1 if b else buf0, sem1 if b else sem0)

    start_dma(0, 0)
    sdiff = zeros
    near = zeros
    for g in range(NCHUNK):
        b = g % 2
        if g + 1 < NCHUNK:
            start_dma(g + 1, (g + 1) % 2)
        pltpu.make_async_copy(
            w_hbm.at[pl.ds(base + g * CHUNK, CHUNK)], buf1 if b else buf0,
            sem1 if b else sem0).wait()
        bufb = buf1 if b else buf0

        @plsc.parallel_loop(0, VPC, step=1, unroll=8, carry=(sdiff, near))
        def chunk_body(i, c):
            sd, nr = c
            w = bufb[pl.ds(i * L, L)]
            t = jnp.minimum(w * ia16, hi)
            t = jnp.maximum(t, lo)
            b16 = (t + magic) - magic
            wq = b16 * a16
            d = jnp.abs(w - wq)
            w2 = w * w
            ai = (b16 + lane).astype(jnp.int32)
            plsc.addupdate_scatter(cnt_t, [ai], ones)
            plsc.addupdate_scatter(sum_t, [ai], w)
            plsc.addupdate_scatter(ssq_t, [ai], w2)
            nr = nr + jnp.where(d < thr, 1.0, 0.0).astype(jnp.float32)
            return (sd + d, nr)

        sdiff, near = chunk_body

    acc[pl.ds(0, L)] = sdiff
    acc[pl.ds(L, L)] = near

    pltpu.sync_copy(cnt_t, out_hbm.at[wid, pl.ds(0, 64)])
    pltpu.sync_copy(sum_t, out_hbm.at[wid, pl.ds(64, 64)])
    pltpu.sync_copy(ssq_t, out_hbm.at[wid, pl.ds(128, 64)])
    pltpu.sync_copy(acc, out_hbm.at[wid, pl.ds(192, 32)])


@jax.jit
def _run(w_flat, params):
    mesh = plsc.VectorSubcoreMesh(core_axis_name="c", subcore_axis_name="s")
    return pl.kernel(
        _body,
        out_type=jax.ShapeDtypeStruct((NW, 224), jnp.float32),
        mesh=mesh,
        compiler_params=pltpu.CompilerParams(needs_layout_passes=False),
        scratch_types=[
            pltpu.VMEM((CHUNK,), jnp.float32),
            pltpu.VMEM((CHUNK,), jnp.float32),
            pltpu.VMEM((4 * L,), jnp.float32),
            pltpu.VMEM((4 * L,), jnp.float32),
            pltpu.VMEM((4 * L,), jnp.float32),
            pltpu.VMEM((4 * L,), jnp.float32),
            pltpu.VMEM((2 * L,), jnp.float32),
            pltpu.SemaphoreType.DMA,
            pltpu.SemaphoreType.DMA,
        ],
    )(w_flat, params)


def kernel(weights, alpha):
    w = weights.reshape(-1)
    a = alpha.reshape(())
    a_s = lax.stop_gradient(a)
    params = jnp.concatenate([
        jnp.full((L,), 16.0, jnp.float32) / a_s,
        jnp.full((L,), 1.0 / 16.0, jnp.float32) * a_s,
        jnp.full((L,), 0.01, jnp.float32) * a_s,
        jnp.arange(L, dtype=jnp.float32) + 32.0,
    ])
    part = _run(w, params)

    cnt = part[:, 0:64].reshape(NW, 4, L).sum(axis=(0, 2))
    sums = part[:, 64:128].reshape(NW, 4, L).sum(axis=(0, 2))
    ssq = part[:, 128:192].reshape(NW, 4, L).sum(axis=(0, 2))
    sdiff = part[:, 192:208].sum()
    near = part[:, 208:224].sum()

    levels = jnp.arange(-2, 2, dtype=jnp.float32) * a_s
    safe_counts = jnp.maximum(cnt, 1.0)
    means = sums / safe_counts
    mse_per_bin = jnp.where(cnt > 0, (means - levels) ** 2, 0.0)
    var_per_bin = jnp.where(cnt >= 2.0, ssq / safe_counts - means ** 2, 0.0)
    total_mse = jnp.sum(mse_per_bin)
    total_var = jnp.sum(var_per_bin)
    loss = total_mse + total_var

    n = jnp.float32(N_TOTAL)
    sumdiff2 = jnp.sum(ssq - 2.0 * levels * sums + levels * levels * cnt)
    quantization_mse = sumdiff2 / n
    mean_distance = sdiff / n
    max_dist = a_s * 0.5
    effectiveness = jnp.clip(
        100.0 * (1.0 - mean_distance / (max_dist + 1e-12)), 0.0, 100.0)
    near_levels = near / n * 100.0

    return (loss, total_mse, total_var, quantization_mse, mean_distance,
            effectiveness, near_levels)
